# baseline (speedup 1.0000x reference)
import jax
import jax.numpy as jnp
from jax import lax
from jax.experimental import pallas as pl
from jax.experimental.pallas import tpu as pltpu
from jax.experimental.pallas import tpu_sc as plsc

N_NODES = 10000
N_EDGES = 320000
D = 128

NC = 2
NS = 16
L = 16

CHUNK = 128
BLK = 8
NBLK = 10
CH_PER_TILE = BLK * NBLK
E_PAD = NC * NS * CH_PER_TILE * CHUNK
ROWS_2D = E_PAD // CHUNK

N_PAD = 10240
ROW_CHUNK = 128
ROW_CHUNKS_PER_SUB = N_PAD // NS // ROW_CHUNK


def _make_edge_agg(use_cids: bool):
    mesh = plsc.VectorSubcoreMesh(
        core_axis_name="c", subcore_axis_name="s", num_cores=NC, num_subcores=NS
    )

    scratch = [
        pltpu.VMEM_SHARED((N_PAD, D), jnp.float32),
        pltpu.VMEM((BLK, CHUNK), jnp.int32),
        pltpu.VMEM((BLK, CHUNK), jnp.int32),
        pltpu.VMEM((BLK, CHUNK), jnp.int32),
        pltpu.VMEM((BLK, CHUNK), jnp.int32),
        pltpu.VMEM((CHUNK, D), jnp.float32),
        pltpu.VMEM((CHUNK, D), jnp.float32),
        pltpu.SemaphoreType.DMA,
        pltpu.SemaphoreType.DMA,
        pltpu.SemaphoreType.DMA,
        pltpu.SemaphoreType.DMA,
        pltpu.SemaphoreType.DMA,
    ]
    if use_cids:
        scratch.insert(1, pltpu.VMEM((N_NODES,), jnp.int32))

    def body(*refs):
        if use_cids:
            (table, src, dst, cids, zeros, out, acc, cncpt_v,
             srcX, srcY, dstX, dstY, rowA, rowB, gsA, gsB, ssA, ssB, isem) = refs
        else:
            (table, src, dst, zeros, out, acc,
             srcX, srcY, dstX, dstY, rowA, rowB, gsA, gsB, ssA, ssB, isem) = refs

        c = lax.axis_index("c")
        s = lax.axis_index("s")
        t = c * NS + s
        row_base = t * CH_PER_TILE

        for k in range(ROW_CHUNKS_PER_SUB):
            row0 = (s * ROW_CHUNKS_PER_SUB + k) * ROW_CHUNK
            pltpu.sync_copy(zeros, acc.at[pl.ds(row0, ROW_CHUNK)])
        if use_cids:
            pltpu.sync_copy(cids, cncpt_v)

        def fire_blk(b, sblk, dblk):
            r0 = row_base + b * BLK
            pltpu.async_copy(src.at[pl.ds(r0, BLK)], sblk, isem)
            pltpu.async_copy(dst.at[pl.ds(r0, BLK)], dblk, isem)

        def drain_blk(b, sblk, dblk):
            r0 = row_base + b * BLK
            pltpu.make_async_copy(src.at[pl.ds(r0, BLK)], sblk, isem).wait()
            pltpu.make_async_copy(dst.at[pl.ds(r0, BLK)], dblk, isem).wait()

        def translate(sblk, p):
            if use_cids:
                for kk in range(CHUNK // L):
                    sl = pl.ds(kk * L, L)
                    sblk[p, sl] = plsc.load_gather(cncpt_v, [sblk[p, sl]])

        def fire_g(sblk, p, buf, gsem):
            pltpu.async_copy(table.at[sblk.at[p]], buf, gsem)

        def drain_g(sblk, p, buf, gsem):
            pltpu.make_async_copy(table.at[sblk.at[p]], buf, gsem).wait()

        r0 = row_base
        pltpu.sync_copy(src.at[pl.ds(r0, BLK)], srcX)
        pltpu.sync_copy(dst.at[pl.ds(r0, BLK)], dstX)
        fire_blk(1, srcY, dstY)
        translate(srcX, 0)
        fire_g(srcX, 0, rowA, gsA)

        plsc.subcore_barrier()

        def block_body(q0, b, cur_s, cur_d, nxt_s, nxt_d):
            for p in range(BLK):
                q = q0 + p
                bufq, gsq, ssq = (rowA, gsA, ssA) if p % 2 == 0 else (rowB, gsB, ssB)
                bufn, gsn, ssn = (rowB, gsB, ssB) if p % 2 == 0 else (rowA, gsA, ssA)

                @pl.when(q + 1 < CH_PER_TILE)
                def _():
                    @pl.when(q >= 1)
                    def _():
                        pltpu.make_async_copy(
                            bufn, acc.at[cur_d.at[max(p - 1, 0)]], ssn
                        ).wait()

                    if p == 0:
                        @pl.when(
                            jnp.logical_and(q0 >= BLK,
                                            q0 + 2 * BLK <= CH_PER_TILE)
                        )
                        def _():
                            fire_blk(b + 1, nxt_s, nxt_d)

                    if p < BLK - 1:
                        translate(cur_s, p + 1)
                        fire_g(cur_s, p + 1, bufn, gsn)
                    else:
                        drain_blk(b + 1, nxt_s, nxt_d)
                        translate(nxt_s, 0)
                        fire_g(nxt_s, 0, bufn, gsn)

                drain_g(cur_s, p, bufq, gsq)
                pltpu.async_copy(bufq, acc.at[cur_d.at[p]], ssq, add=True)

        def outer(io, carry):
            q0 = io * 2 * BLK
            b = io * 2
            block_body(q0, b, srcX, dstX, srcY, dstY)
            block_body(q0 + BLK, b + 1, srcY, dstY, srcX, dstX)
            return carry

        lax.fori_loop(0, NBLK // 2, outer, 0)

        pltpu.make_async_copy(rowA, acc.at[dstY.at[BLK - 2]], ssA).wait()
        pltpu.make_async_copy(rowB, acc.at[dstY.at[BLK - 1]], ssB).wait()

        plsc.subcore_barrier()

        for k in range(ROW_CHUNKS_PER_SUB):
            row0 = (s * ROW_CHUNKS_PER_SUB + k) * ROW_CHUNK
            pltpu.sync_copy(acc.at[pl.ds(row0, ROW_CHUNK)], out.at[c, pl.ds(row0, ROW_CHUNK)])

    return pl.kernel(
        body,
        out_type=jax.ShapeDtypeStruct((NC, N_PAD, D), jnp.float32),
        mesh=mesh,
        scratch_types=scratch,
        compiler_params=pltpu.CompilerParams(needs_layout_passes=False),
        name="edge_agg_cids" if use_cids else "edge_agg",
    )


def _linear_relu_body(p_ref, w_ref, b_ref, o_ref):
    x = p_ref[0] + p_ref[1]
    y = jnp.dot(x, w_ref[...], preferred_element_type=jnp.float32) + b_ref[...]
    o_ref[...] = jnp.maximum(y, 0.0)


def _linear_relu(parts, W, b):
    BN = 2000
    return pl.pallas_call(
        _linear_relu_body,
        grid=(N_NODES // BN,),
        in_specs=[
            pl.BlockSpec((NC, BN, D), lambda i: (0, i, 0)),
            pl.BlockSpec((D, D), lambda i: (0, 0)),
            pl.BlockSpec((1, D), lambda i: (0, 0)),
        ],
        out_specs=pl.BlockSpec((BN, D), lambda i: (i, 0)),
        out_shape=jax.ShapeDtypeStruct((N_NODES, D), jnp.float32),
    )(parts, W, b.reshape(1, D))


@jax.jit
def kernel(cncpt_ids, edge_index, emb, W1, b1, W2, b2):
    npad = E_PAD - N_EDGES
    src = jnp.concatenate([edge_index[0], jnp.zeros((npad,), jnp.int32)])
    dst = jnp.concatenate([edge_index[1], jnp.full((npad,), N_NODES, jnp.int32)])
    src2d = src.reshape(ROWS_2D, CHUNK)
    dst2d = dst.reshape(ROWS_2D, CHUNK)
    zeros = jnp.zeros((ROW_CHUNK, D), jnp.float32)

    agg1 = _make_edge_agg(True)(emb, src2d, dst2d, cncpt_ids, zeros)
    h1 = _linear_relu(agg1, W1, b1)
    agg2 = _make_edge_agg(False)(h1, src2d, dst2d, zeros)
    h2 = _linear_relu(agg2, W2, b2)
    return h2
